# baseline (speedup 1.0000x reference)
import jax
import jax.numpy as jnp
from jax.experimental import pallas as pl

_S = 4
_GRID = 8


def _route(lt):
    def row(k):
        return lt[k:k + 1, :]

    part = row(1) > row(0)
    bv = row(2)
    bi = jnp.zeros_like(bv, dtype=jnp.int32)
    t = row(3) > bv
    bi = jnp.where(t, 1, bi)
    bv = jnp.where(t, row(3), bv)
    t = row(4) > bv
    bi = jnp.where(t, 2, bi)
    rv = row(5)
    ri = jnp.zeros_like(bv, dtype=jnp.int32)
    for k in range(1, 8):
        t = row(5 + k) > rv
        ri = jnp.where(t, k, ri)
        rv = jnp.where(t, row(5 + k), rv)
    a0 = row(13) >= row(14)

    branch = jnp.where(bi == 0, 3, jnp.where(bi == 1, ri + 1, jnp.where(a0, 4, 6)))
    return jnp.where(part, branch, 0).astype(jnp.int32)


def _body(*refs):
    x_refs = refs[:_S]
    w_ref, b_ref, o_ref = refs[_S], refs[_S + 1], refs[_S + 2]
    for s in range(_S):
        l = jnp.dot(x_refs[s][...], w_ref[...], preferred_element_type=jnp.float32)
        l = l + b_ref[...]
        o_ref[0, s, :] = _route(l.T)[0, :]


def kernel(x, W1, b1, W2, b2, W3r, b3r, W3a, b3a):
    batch = x.shape[0]
    xf = x.reshape(batch, -1)
    d = xf.shape[1]
    W = jnp.concatenate([W1, W2, W3r, W3a], axis=1)
    b = jnp.concatenate([b1, b2, b3r, b3a], axis=0)
    W = jnp.pad(W, ((0, 0), (0, 128 - W.shape[1])))
    b = jnp.pad(b, ((0, 128 - b.shape[0]),)).reshape(1, 128)

    chunk = batch // _S
    bs = chunk // _GRID
    xs = [jax.lax.slice(xf, (s * chunk, 0), ((s + 1) * chunk, d)) for s in range(_S)]

    out = pl.pallas_call(
        _body,
        grid=(_GRID,),
        in_specs=[pl.BlockSpec((bs, d), lambda i: (i, 0)) for _ in range(_S)]
        + [
            pl.BlockSpec((d, 128), lambda i: (0, 0)),
            pl.BlockSpec((1, 128), lambda i: (0, 0)),
        ],
        out_specs=pl.BlockSpec((1, _S, bs), lambda i: (i, 0, 0)),
        out_shape=jax.ShapeDtypeStruct((_GRID, _S, bs), jnp.int32),
    )(*xs, W, b)
    return out.transpose(1, 0, 2).reshape(batch)
